# Initial kernel scaffold; baseline (speedup 1.0000x reference)
import jax
import jax.numpy as jnp
from jax.experimental import pallas as pl


def kernel(loc_preds, loc_targets, cls_preds, cls_targets, alpha):
    raise NotImplementedError("write your pallas kernel here")


# speedup vs baseline: 19.5893x; 19.5893x over previous
import functools

import jax
import jax.numpy as jnp
from jax import lax
from jax.experimental import pallas as pl
from jax.experimental.pallas import tpu as pltpu

_C = 21
_LANES = 128
_ROWS_PER_CHUNK = 112
_GRID = 7


def _rrd_body(alpha_ref, cls_ref, tgt_ref, lp_ref, lt_ref, out_ref,
              vscr, acc_loc, acc_pos, acc_pcls, *, total_rows, n_anchors_pad):
    g = pl.program_id(0)

    @pl.when(g == 0)
    def _init():
        z = jnp.zeros((_ROWS_PER_CHUNK, _LANES), jnp.float32)
        acc_loc[...] = z
        acc_pos[...] = z
        acc_pcls[...] = z

    x = cls_ref[...]
    t = tgt_ref[...]
    m = jnp.max(x, axis=0)
    e = jnp.exp(x - m[None, :, :])
    s = jnp.sum(e, axis=0)
    lse = m + jnp.log(s)
    ci = lax.broadcasted_iota(jnp.int32, x.shape, 0)
    xt = jnp.sum(jnp.where(ci == t[None, :, :], x, 0.0), axis=0)
    valid = t >= 0
    pos = t > 0
    ce = jnp.where(valid, lse - xt, 0.0)
    posf = pos.astype(jnp.float32)
    v = jnp.where(valid & jnp.logical_not(pos), jnp.maximum(ce, 0.0), 0.0)

    d = lp_ref[...] - lt_ref[...]
    ad = jnp.abs(d)
    sl1 = jnp.where(ad < 1.0, 0.5 * d * d, ad - 0.5)

    acc_loc[...] += jnp.sum(sl1, axis=0) * posf
    acc_pos[...] += posf
    acc_pcls[...] += ce * posf
    vscr[pl.ds(g * _ROWS_PER_CHUNK, _ROWS_PER_CHUNK), :] = v

    @pl.when(g == pl.num_programs(0) - 1)
    def _final():
        np_f = jnp.sum(acc_pos[...])
        np_i = np_f.astype(jnp.int32)
        k_neg = jnp.where(np_i > 0, 3 * np_i, 10)
        k_c = jnp.minimum(k_neg, n_anchors_pad)
        vall = vscr[...]
        u = lax.bitcast_convert_type(vall, jnp.int32)

        def bit_body(i, prefix):
            cand = prefix | lax.shift_left(jnp.int32(1), 30 - i)
            cnt = jnp.sum((u >= cand).astype(jnp.int32))
            return jnp.where(cnt >= k_c, cand, prefix)

        tb = lax.fori_loop(0, 31, bit_body, jnp.int32(0))
        t_f = lax.bitcast_convert_type(tb, jnp.float32)
        gt = u > tb
        cnt_gt = jnp.sum(gt.astype(jnp.int32))
        sum_gt = jnp.sum(jnp.where(gt, vall, 0.0))
        extra = sum_gt + (k_c - cnt_gt).astype(jnp.float32) * t_f
        num = (alpha_ref[0, 0] * jnp.sum(acc_loc[...])
               + jnp.sum(acc_pcls[...]) + extra)
        den = np_f + k_neg.astype(jnp.float32)
        out_ref[0, 0] = num / den


def _rrd_loss(loc_preds, loc_targets, cls_preds, cls_targets, alpha,
              interpret=False):
    n, a, c = cls_preds.shape
    chunk = _ROWS_PER_CHUNK * _LANES
    a_pad = ((a + chunk - 1) // chunk) * chunk
    total_rows = a_pad // _LANES
    grid = total_rows // _ROWS_PER_CHUNK

    cls_t = jnp.pad(cls_preds.reshape(a, c).T, ((0, 0), (0, a_pad - a)))
    cls3 = cls_t.reshape(c, total_rows, _LANES)
    tgt = jnp.pad(cls_targets.reshape(a).astype(jnp.int32), (0, a_pad - a),
                  constant_values=-1).reshape(total_rows, _LANES)
    lp3 = jnp.pad(loc_preds.reshape(a, 8).T, ((0, 0), (0, a_pad - a))
                  ).reshape(8, total_rows, _LANES)
    lt3 = jnp.pad(loc_targets.reshape(a, 8).T, ((0, 0), (0, a_pad - a))
                  ).reshape(8, total_rows, _LANES)
    alpha_s = jnp.asarray(alpha, jnp.float32).reshape(1, 1)

    body = functools.partial(_rrd_body, total_rows=total_rows,
                             n_anchors_pad=a_pad)
    out = pl.pallas_call(
        body,
        grid=(grid,),
        in_specs=[
            pl.BlockSpec(memory_space=pltpu.SMEM),
            pl.BlockSpec((c, _ROWS_PER_CHUNK, _LANES), lambda g: (0, g, 0)),
            pl.BlockSpec((_ROWS_PER_CHUNK, _LANES), lambda g: (g, 0)),
            pl.BlockSpec((8, _ROWS_PER_CHUNK, _LANES), lambda g: (0, g, 0)),
            pl.BlockSpec((8, _ROWS_PER_CHUNK, _LANES), lambda g: (0, g, 0)),
        ],
        out_specs=pl.BlockSpec(memory_space=pltpu.SMEM),
        out_shape=jax.ShapeDtypeStruct((1, 1), jnp.float32),
        scratch_shapes=[
            pltpu.VMEM((total_rows, _LANES), jnp.float32),
            pltpu.VMEM((_ROWS_PER_CHUNK, _LANES), jnp.float32),
            pltpu.VMEM((_ROWS_PER_CHUNK, _LANES), jnp.float32),
            pltpu.VMEM((_ROWS_PER_CHUNK, _LANES), jnp.float32),
        ],
        compiler_params=pltpu.CompilerParams(
            dimension_semantics=("arbitrary",),
        ),
        interpret=interpret,
    )(alpha_s, cls3, tgt, lp3, lt3)
    return out.reshape((1,))


def kernel(loc_preds, loc_targets, cls_preds, cls_targets, alpha):
    return _rrd_loss(loc_preds, loc_targets, cls_preds, cls_targets, alpha)
